# baseline (speedup 1.0000x reference)
import functools

import jax
import jax.numpy as jnp
from jax import lax
from jax.experimental import pallas as pl
from jax.experimental.pallas import tpu as pltpu
from jax.experimental.pallas import tpu_sc as plsc

B, N, DF, K = 4, 2048, 128, 16
RB = 256
NB = N // RB
BN = B * N

NW = 32
ROWS_W = N // NW
RG = 16
NGRP = ROWS_W // RG
GCAP = 32
CCAP = 128

NEG = float("-inf")


def _embed_body(x_ref, w_ref, xe_ref, xc_ref, xct_ref):
    x = x_ref[0]
    w = w_ref[...]
    xe = jnp.dot(x, w, preferred_element_type=jnp.float32)
    xe_ref[0] = xe
    mean = jnp.mean(xe, axis=0, keepdims=True)
    xc = xe - mean
    xc_ref[0] = xc
    xct_ref[0] = xc.T


def _lq_body(scale_ref, xcr_ref, xct_ref, q_ref, lq_ref, gmax_ref, tau_ref):
    r0 = pl.program_id(0) * RB
    xr = xcr_ref[...]
    xt = xct_ref[...]
    s = jnp.dot(xr, xt, preferred_element_type=jnp.float32)
    x2r = jnp.sum(xr * xr, axis=1, keepdims=True)
    x2c = jnp.sum(xt * xt, axis=0, keepdims=True)
    d = jnp.maximum(x2r + x2c - 2.0 * s, 0.0)
    scale = scale_ref[0]
    col = lax.broadcasted_iota(jnp.int32, (RB, N), 1)
    row = lax.broadcasted_iota(jnp.int32, (RB, N), 0) + r0
    g = jnp.log(-jnp.log(q_ref[...]))
    vals = jnp.where(col == row, NEG, -d * scale - g)
    lq_ref[...] = vals.reshape(RB // 8, 8, 16, 128).transpose(0, 2, 1, 3)
    gm = vals[:, 0:128]
    for t in range(1, 16):
        gm = jnp.maximum(gm, vals[:, t * 128 : (t + 1) * 128])
    gmax_ref[...] = gm
    g2 = gm
    for _ in range(15):
        m = jnp.max(g2, axis=1, keepdims=True)
        g2 = jnp.where(g2 == m, NEG, g2)
    tau_ref[...] = jnp.max(g2, axis=1, keepdims=True).reshape(RB)


def _sc_topk(lq_hbm, gmax_hbm, tau_hbm, lp_hbm, idx_hbm,
             rowbs, gbs, tbs, lpbs, idxbs, gidb, candv, candp,
             insems, outsems):
    wid = lax.axis_index("s") * 2 + lax.axis_index("c")
    wrow0 = wid * ROWS_W
    lane = lax.iota(jnp.int32, 16)
    ninf = jnp.full((16,), NEG, jnp.float32)

    def fetch(gidx, p):
        row0 = wrow0 + gidx * RG
        pltpu.async_copy(lq_hbm.at[pl.ds(row0 * N, RG * N)], rowbs[p],
                         insems[p])
        pltpu.async_copy(gmax_hbm.at[pl.ds(row0 * 128, RG * 128)], gbs[p],
                         insems[p])
        pltpu.async_copy(tau_hbm.at[pl.ds(row0, RG)], tbs[p], insems[p])

    def drain_in(gidx, p):
        row0 = wrow0 + gidx * RG
        pltpu.make_async_copy(lq_hbm.at[pl.ds(row0 * N, RG * N)], rowbs[p],
                              insems[p]).wait()
        pltpu.make_async_copy(gmax_hbm.at[pl.ds(row0 * 128, RG * 128)],
                              gbs[p], insems[p]).wait()
        pltpu.make_async_copy(tau_hbm.at[pl.ds(row0, RG)], tbs[p],
                              insems[p]).wait()

    def merge_body(m, carry):
        rv, ri, ccs = carry
        valid = (lane + m * 16) < ccs
        c = jnp.where(valid, candv[pl.ds(m * 16, 16)], ninf)
        p = candp[pl.ds(m * 16, 16)]
        cs, cp = plsc.sort_key_val(c, p, descending=True)
        csr = jnp.flip(cs, 0)
        cpr = jnp.flip(cp, 0)
        take = rv >= csr
        nv = jnp.where(take, rv, csr)
        np_ = jnp.where(take, ri, cpr)
        rv, ri = plsc.sort_key_val(nv, np_, descending=True)
        return rv, ri, ccs

    def process(p):
        rowb, gb, tb, lpb, idxb = rowbs[p], gbs[p], tbs[p], lpbs[p], idxbs[p]

        def row_body(i, _):
            tsplat = plsc.load_gather(tb, [jnp.full((16,), i, jnp.int32)])
            ibase = (i // 8) * 16384 + (i % 8) * 128

            @plsc.parallel_loop(0, 8, 1, unroll=4,
                                carry=jnp.zeros((16,), jnp.int32))
            def gcnt(j, cur):
                gm = gb[pl.ds(i * 128 + j * 16, 16)]
                msk = gm >= tsplat
                pos = cur + plsc.cumsum(msk.astype(jnp.int32)) - 1
                msk = msk & (pos < GCAP)
                plsc.store_scatter(gidb, [pos], lane + j * 16, mask=msk)
                return cur + plsc.all_reduce_population_count(msk)

            def round_fn(r, ccur):
                gids = gidb[pl.ds(r * 16, 16)]
                gvalid = (lane + r * 16) < gcnt
                gids = jnp.where(gvalid, gids, 0)

                @plsc.parallel_loop(0, 16, 1, unroll=4, carry=ccur)
                def cc_out(t, cc):
                    colidx = gids + t * 128
                    v = plsc.load_gather(rowb, [gids + (t * 1024 + ibase)],
                                         mask=gvalid)
                    msk = gvalid & (v >= tsplat)
                    pos = cc + plsc.cumsum(msk.astype(jnp.int32)) - 1
                    msk = msk & (pos < CCAP)
                    plsc.store_scatter(candv, [pos], v, mask=msk)
                    plsc.store_scatter(candp, [pos], colidx, mask=msk)
                    return cc + plsc.all_reduce_population_count(msk)
                return cc_out

            ccur = round_fn(0, jnp.zeros((16,), jnp.int32))
            gs = jnp.max(gcnt)
            ccur = lax.cond(gs > 16, lambda c: round_fn(1, c),
                            lambda c: c, ccur)

            nmerge = (jnp.max(ccur) + 15) // 16
            rv, ri, _ = lax.fori_loop(
                0, nmerge, merge_body,
                (ninf, jnp.zeros((16,), jnp.int32), ccur))
            lpb[pl.ds(i * 16, 16)] = rv
            idxb[pl.ds(i * 16, 16)] = ri
            return 0

        lax.fori_loop(0, RG, row_body, 0)

    def put(gidx, p):
        row0 = wrow0 + gidx * RG
        pltpu.async_copy(lpbs[p], lp_hbm.at[pl.ds(row0 * K, RG * K)],
                         outsems[p])
        pltpu.async_copy(idxbs[p], idx_hbm.at[pl.ds(row0 * K, RG * K)],
                         outsems[p])

    def drain_out(gidx, p):
        row0 = wrow0 + gidx * RG
        pltpu.make_async_copy(lpbs[p], lp_hbm.at[pl.ds(row0 * K, RG * K)],
                              outsems[p]).wait()
        pltpu.make_async_copy(idxbs[p], idx_hbm.at[pl.ds(row0 * K, RG * K)],
                              outsems[p]).wait()

    fetch(0, 0)

    def group_pair(g2, _):
        g0 = g2 * 2
        for p in range(2):
            g = g0 + p
            drain_in(g, p)
            nxt = jnp.minimum(g + 1, NGRP - 1)

            @pl.when(g + 1 < NGRP)
            def _():
                fetch(nxt, 1 - p)

            @pl.when(g >= 2)
            def _():
                drain_out(g - 2, p)

            process(p)
            put(g, p)
        return 0

    lax.fori_loop(0, NGRP // 2, group_pair, 0)
    drain_out(NGRP - 2, 0)
    drain_out(NGRP - 1, 1)


_sc_topk_call = functools.partial(
    pl.kernel,
    out_type=[
        jax.ShapeDtypeStruct((N * K,), jnp.float32),
        jax.ShapeDtypeStruct((N * K,), jnp.int32),
    ],
    mesh=plsc.VectorSubcoreMesh(core_axis_name="c", subcore_axis_name="s"),
    compiler_params=pltpu.CompilerParams(needs_layout_passes=False),
    scratch_types=[
        [pltpu.VMEM((RG * N,), jnp.float32)] * 2,
        [pltpu.VMEM((RG * 128,), jnp.float32)] * 2,
        [pltpu.VMEM((RG,), jnp.float32)] * 2,
        [pltpu.VMEM((RG * K,), jnp.float32)] * 2,
        [pltpu.VMEM((RG * K,), jnp.int32)] * 2,
        pltpu.VMEM((GCAP,), jnp.int32),
        pltpu.VMEM((CCAP,), jnp.float32),
        pltpu.VMEM((CCAP,), jnp.int32),
        [pltpu.SemaphoreType.DMA] * 2,
        [pltpu.SemaphoreType.DMA] * 2,
    ],
)(_sc_topk)


@jax.jit
def kernel(x, A, W, temperature, q):
    del A
    scale = jnp.exp(jnp.clip(temperature, -4.0, 4.0)).reshape(1)

    xe, xc, xct = pl.pallas_call(
        _embed_body,
        grid=(B,),
        in_specs=[
            pl.BlockSpec((1, N, DF), lambda b: (b, 0, 0)),
            pl.BlockSpec((DF, DF), lambda b: (0, 0)),
        ],
        out_specs=[
            pl.BlockSpec((1, N, DF), lambda b: (b, 0, 0)),
            pl.BlockSpec((1, N, DF), lambda b: (b, 0, 0)),
            pl.BlockSpec((1, DF, N), lambda b: (b, 0, 0)),
        ],
        out_shape=[
            jax.ShapeDtypeStruct((B, N, DF), jnp.float32),
            jax.ShapeDtypeStruct((B, N, DF), jnp.float32),
            jax.ShapeDtypeStruct((B, DF, N), jnp.float32),
        ],
    )(x, W)

    lq_call = pl.pallas_call(
        _lq_body,
        grid=(NB,),
        in_specs=[
            pl.BlockSpec(memory_space=pltpu.SMEM),
            pl.BlockSpec((RB, DF), lambda r: (r, 0)),
            pl.BlockSpec((DF, N), lambda r: (0, 0)),
            pl.BlockSpec((RB, N), lambda r: (r, 0)),
        ],
        out_specs=[
            pl.BlockSpec((RB // 8, 16, 8, 128), lambda r: (r, 0, 0, 0)),
            pl.BlockSpec((RB, 128), lambda r: (r, 0)),
            pl.BlockSpec((RB,), lambda r: (r,)),
        ],
        out_shape=[
            jax.ShapeDtypeStruct((N // 8, 16, 8, 128), jnp.float32),
            jax.ShapeDtypeStruct((N, 128), jnp.float32),
            jax.ShapeDtypeStruct((N,), jnp.float32),
        ],
    )

    lps, idxs = [], []
    for b in range(B):
        lq_b, gmax_b, tau_b = lq_call(scale, xc[b], xct[b], q[b])
        lp_b, idx_b = _sc_topk_call(
            lq_b.reshape(-1), gmax_b.reshape(-1), tau_b
        )
        lps.append(lp_b)
        idxs.append(idx_b)
    lp = jnp.stack(lps).reshape(B, N, K)
    idx = jnp.stack(idxs).reshape(B, N, K)

    offs = (jnp.arange(B, dtype=jnp.int32) * N)[:, None]
    e0 = idx.reshape(B, N * K) + offs
    e1 = jnp.repeat(jnp.arange(N, dtype=jnp.int32), K)[None, :] + offs
    edges_sparse = jnp.stack((e0, e1), 0).reshape(2, -1)
    return xe, edges_sparse, lp
